# Initial kernel scaffold; baseline (speedup 1.0000x reference)
import jax
import jax.numpy as jnp
from jax.experimental import pallas as pl


def kernel(norms):
    raise NotImplementedError("write your pallas kernel here")


# speedup vs baseline: 2.1808x; 2.1808x over previous
import functools

import jax
import jax.numpy as jnp
from jax import lax
from jax.experimental import pallas as pl
from jax.experimental.pallas import tpu as pltpu
from jax.experimental.pallas import tpu_sc as plsc

_NL = 16
_RADIX = 256
_PASSES = 4


def _row_sort_and_label(vbuf, key_a, pay_a, key_b, pay_b, hist, obuf, S):
    NV = S // _NL
    NH = _RADIX // _NL

    def key_init(j, _):
        sl = pl.ds(j * _NL, _NL)
        b = plsc.bitcast(vbuf[sl], jnp.int32)
        flip = jnp.where(b < 0, jnp.int32(-1), jnp.int32(-(2 ** 31)))
        key_a[sl] = b ^ flip
        pay_a[sl] = lax.iota(jnp.int32, _NL) + j * _NL
        return 0

    lax.fori_loop(0, NV, key_init, 0)

    for pidx in range(_PASSES):
        kin, pin = (key_a, pay_a) if pidx % 2 == 0 else (key_b, pay_b)
        kout, pout = (key_b, pay_b) if pidx % 2 == 0 else (key_a, pay_a)
        sh = 8 * pidx

        def zero_body(c, _):
            hist[pl.ds(c * _NL, _NL)] = jnp.zeros((_NL,), jnp.int32)
            return 0

        lax.fori_loop(0, NH, zero_body, 0)

        def hist_body(j, _, kin=kin, sh=sh):
            d = lax.shift_right_logical(kin[pl.ds(j * _NL, _NL)], sh) & 255
            cnt, last = plsc.scan_count(d)
            plsc.addupdate_scatter(hist, [d], cnt, mask=last)
            return 0

        lax.fori_loop(0, NV, hist_body, 0)

        def scan_body(c, carry):
            sl = pl.ds(c * _NL, _NL)
            h = hist[sl]
            hist[sl] = plsc.cumsum(h) - h + carry
            return carry + jnp.sum(h)

        lax.fori_loop(0, NH, scan_body, jnp.int32(0))

        def perm_body(j, _, kin=kin, pin=pin, kout=kout, pout=pout, sh=sh):
            sl = pl.ds(j * _NL, _NL)
            k = kin[sl]
            d = lax.shift_right_logical(k, sh) & 255
            cnt, last = plsc.scan_count(d)
            pos = plsc.load_gather(hist, [d]) + cnt - 1
            plsc.store_scatter(kout, [pos], k)
            plsc.store_scatter(pout, [pos], pin[sl])
            plsc.store_scatter(hist, [d], pos + 1, mask=last)
            return 0

        lax.fori_loop(0, NV, perm_body, 0)

    big = jnp.int32(S + 10)

    def lab_body(j, carry):
        sl = pl.ds(j * _NL, _NL)
        u = key_a[sl]
        flip = jnp.where(u < 0, jnp.int32(-(2 ** 31)), jnp.int32(-1))
        v = plsc.bitcast(u ^ flip, jnp.float32)
        p_vec = lax.iota(jnp.int32, _NL) + j * _NL
        x = jnp.clip(v + 0.5, 2.0, 4096.0)
        a = x.astype(jnp.int32) - p_vec
        a = jnp.where(p_vec >= 2, a, big)
        g = jnp.minimum(-plsc.cummax(-a), carry)
        lab = (p_vec + jnp.minimum(g, 1)).astype(jnp.float32)
        plsc.store_scatter(obuf, [pay_a[sl]], lab)
        return jnp.min(g)

    lax.fori_loop(0, NV, lab_body, big)


def kernel(norms):
    B, S = norms.shape
    mesh = plsc.VectorSubcoreMesh(core_axis_name="c", subcore_axis_name="s")
    n_workers = mesh.num_cores * mesh.num_subcores
    rows_per_w = B // n_workers

    @functools.partial(
        pl.kernel,
        out_type=jax.ShapeDtypeStruct((B, S), jnp.float32),
        mesh=mesh,
        scratch_types=[
            pltpu.VMEM((S,), jnp.float32),
            pltpu.VMEM((S,), jnp.int32),
            pltpu.VMEM((S,), jnp.int32),
            pltpu.VMEM((S,), jnp.int32),
            pltpu.VMEM((S,), jnp.int32),
            pltpu.VMEM((_RADIX,), jnp.int32),
            pltpu.VMEM((S,), jnp.float32),
        ],
        compiler_params=pltpu.CompilerParams(needs_layout_passes=False),
    )
    def sc_kernel(norms_hbm, out_hbm, vbuf, key_a, pay_a, key_b, pay_b, hist,
                  obuf):
        wid = lax.axis_index("s") * mesh.num_cores + lax.axis_index("c")
        row0 = wid * rows_per_w

        def row_body(r, _):
            row = row0 + r
            pltpu.sync_copy(norms_hbm.at[row], vbuf)
            _row_sort_and_label(vbuf, key_a, pay_a, key_b, pay_b, hist, obuf,
                                S)
            pltpu.sync_copy(obuf, out_hbm.at[row])
            return 0

        lax.fori_loop(0, rows_per_w, row_body, 0)

    return sc_kernel(norms)


# speedup vs baseline: 2.3924x; 1.0971x over previous
import functools

import jax
import jax.numpy as jnp
from jax import lax
from jax.experimental import pallas as pl
from jax.experimental.pallas import tpu as pltpu
from jax.experimental.pallas import tpu_sc as plsc

_NL = 16
_RADIX = 2048
_DMASK = _RADIX - 1
_SHIFTS = (0, 11, 22)
_NR = 2


def _rows_sort_and_label(vbuf, key_a, pay_a, key_b, pay_b, hist, obuf, S):
    NV = S // _NL
    NH = _RADIX // _NL

    def key_init(j, _):
        for i in range(_NR):
            sl = pl.ds(i * S + j * _NL, _NL)
            b = plsc.bitcast(vbuf[sl], jnp.int32)
            flip = jnp.where(b < 0, jnp.int32(-1), jnp.int32(-(2 ** 31)))
            key_a[sl] = b ^ flip
            pay_a[sl] = lax.iota(jnp.int32, _NL) + j * _NL
        return 0

    lax.fori_loop(0, NV, key_init, 0)

    for pidx, sh in enumerate(_SHIFTS):
        kin, pin = (key_a, pay_a) if pidx % 2 == 0 else (key_b, pay_b)
        kout, pout = (key_b, pay_b) if pidx % 2 == 0 else (key_a, pay_a)

        def zero_body(c, _):
            for i in range(_NR):
                hist[pl.ds(i * _RADIX + c * _NL, _NL)] = jnp.zeros(
                    (_NL,), jnp.int32)
            return 0

        lax.fori_loop(0, NH, zero_body, 0)

        def hist_body(j, _, kin=kin, sh=sh):
            for i in range(_NR):
                d = lax.shift_right_logical(
                    kin[pl.ds(i * S + j * _NL, _NL)], sh) & _DMASK
                cnt, last = plsc.scan_count(d)
                plsc.addupdate_scatter(hist, [d + i * _RADIX], cnt, mask=last)
            return 0

        lax.fori_loop(0, NV, hist_body, 0)

        def scan_body(c, carry):
            out = []
            for i in range(_NR):
                sl = pl.ds(i * _RADIX + c * _NL, _NL)
                h = hist[sl]
                hist[sl] = plsc.cumsum(h) - h + carry[i]
                out.append(carry[i] + jnp.sum(h))
            return tuple(out)

        lax.fori_loop(0, NH, scan_body, (jnp.int32(0),) * _NR)

        def perm_body(j, _, kin=kin, pin=pin, kout=kout, pout=pout, sh=sh):
            for i in range(_NR):
                sl = pl.ds(i * S + j * _NL, _NL)
                k = kin[sl]
                d = lax.shift_right_logical(k, sh) & _DMASK
                dh = d + i * _RADIX
                cnt, last = plsc.scan_count(d)
                pos = plsc.load_gather(hist, [dh]) + cnt - 1
                plsc.store_scatter(kout, [pos + i * S], k)
                plsc.store_scatter(pout, [pos + i * S], pin[sl])
                plsc.store_scatter(hist, [dh], pos + 1, mask=last)
            return 0

        lax.fori_loop(0, NV, perm_body, 0)

    kfin, pfin = (key_b, pay_b) if len(_SHIFTS) % 2 == 1 else (key_a, pay_a)

    big = jnp.int32(S + 10)

    def lab_body(j, carry):
        out = []
        for i in range(_NR):
            sl = pl.ds(i * S + j * _NL, _NL)
            u = kfin[sl]
            flip = jnp.where(u < 0, jnp.int32(-(2 ** 31)), jnp.int32(-1))
            v = plsc.bitcast(u ^ flip, jnp.float32)
            p_vec = lax.iota(jnp.int32, _NL) + j * _NL
            x = jnp.clip(v + 0.5, 2.0, 4096.0)
            a = x.astype(jnp.int32) - p_vec
            a = jnp.where(p_vec >= 2, a, big)
            g = jnp.minimum(-plsc.cummax(-a), carry[i])
            lab = (p_vec + jnp.minimum(g, 1)).astype(jnp.float32)
            plsc.store_scatter(obuf, [pfin[sl] + i * S], lab)
            out.append(jnp.min(g))
        return tuple(out)

    lax.fori_loop(0, NV, lab_body, (big,) * _NR)


def kernel(norms):
    B, S = norms.shape
    mesh = plsc.VectorSubcoreMesh(core_axis_name="c", subcore_axis_name="s")
    n_workers = mesh.num_cores * mesh.num_subcores
    rows_per_w = B // n_workers

    @functools.partial(
        pl.kernel,
        out_type=jax.ShapeDtypeStruct((B, S), jnp.float32),
        mesh=mesh,
        scratch_types=[
            pltpu.VMEM((_NR * S,), jnp.float32),
            pltpu.VMEM((_NR * S,), jnp.int32),
            pltpu.VMEM((_NR * S,), jnp.int32),
            pltpu.VMEM((_NR * S,), jnp.int32),
            pltpu.VMEM((_NR * S,), jnp.int32),
            pltpu.VMEM((_NR * _RADIX,), jnp.int32),
            pltpu.VMEM((_NR * S,), jnp.float32),
        ],
        compiler_params=pltpu.CompilerParams(needs_layout_passes=False),
    )
    def sc_kernel(norms_hbm, out_hbm, vbuf, key_a, pay_a, key_b, pay_b, hist,
                  obuf):
        wid = lax.axis_index("s") * mesh.num_cores + lax.axis_index("c")
        row0 = wid * rows_per_w

        def row_body(r, _):
            row = row0 + r * _NR
            for i in range(_NR):
                pltpu.sync_copy(norms_hbm.at[row + i],
                                vbuf.at[pl.ds(i * S, S)])
            _rows_sort_and_label(vbuf, key_a, pay_a, key_b, pay_b, hist,
                                 obuf, S)
            for i in range(_NR):
                pltpu.sync_copy(obuf.at[pl.ds(i * S, S)],
                                out_hbm.at[row + i])
            return 0

        lax.fori_loop(0, rows_per_w // _NR, row_body, 0)

    return sc_kernel(norms)


# speedup vs baseline: 2.8776x; 1.2028x over previous
import functools

import jax
import jax.numpy as jnp
from jax import lax
from jax.experimental import pallas as pl
from jax.experimental.pallas import tpu as pltpu
from jax.experimental.pallas import tpu_sc as plsc

_NL = 16
_RADIX = 2048
_DMASK = _RADIX - 1
_SHIFTS = (0, 11, 22)
_NR = 2
_UNROLL = 4


def _rows_sort_and_label(rows, S):
    NV = S // _NL
    NH = _RADIX // _NL

    def key_init(jj, _):
        for u in range(_UNROLL):
            j = jj * _UNROLL + u
            sl = pl.ds(j * _NL, _NL)
            for (vbuf, key_a, pay_a, _kb, _pb, _h, _o) in rows:
                b = plsc.bitcast(vbuf[sl], jnp.int32)
                flip = jnp.where(b < 0, jnp.int32(-1), jnp.int32(-(2 ** 31)))
                key_a[sl] = b ^ flip
                pay_a[sl] = lax.iota(jnp.int32, _NL) + j * _NL
        return 0

    lax.fori_loop(0, NV // _UNROLL, key_init, 0)

    for pidx, sh in enumerate(_SHIFTS):
        if pidx % 2 == 0:
            ios = [(r[1], r[2], r[3], r[4], r[5]) for r in rows]
        else:
            ios = [(r[3], r[4], r[1], r[2], r[5]) for r in rows]

        def zero_body(cc, _, ios=ios):
            for u in range(_UNROLL):
                sl = pl.ds((cc * _UNROLL + u) * _NL, _NL)
                for (_ki, _pi, _ko, _po, hist) in ios:
                    hist[sl] = jnp.zeros((_NL,), jnp.int32)
            return 0

        lax.fori_loop(0, NH // _UNROLL, zero_body, 0)

        def hist_body(jj, _, ios=ios, sh=sh):
            for u in range(_UNROLL):
                sl = pl.ds((jj * _UNROLL + u) * _NL, _NL)
                for (kin, _pi, _ko, _po, hist) in ios:
                    d = lax.shift_right_logical(kin[sl], sh) & _DMASK
                    cnt, last = plsc.scan_count(d)
                    plsc.addupdate_scatter(hist, [d], cnt, mask=last)
            return 0

        lax.fori_loop(0, NV // _UNROLL, hist_body, 0)

        def scan_body(cc, carry, ios=ios):
            carry = list(carry)
            for u in range(_UNROLL):
                sl = pl.ds((cc * _UNROLL + u) * _NL, _NL)
                for i, (_ki, _pi, _ko, _po, hist) in enumerate(ios):
                    h = hist[sl]
                    cs = plsc.cumsum(h)
                    hist[sl] = cs - h + carry[i]
                    carry[i] = carry[i] + cs[15]
            return tuple(carry)

        lax.fori_loop(0, NH // _UNROLL, scan_body, (jnp.int32(0),) * _NR)

        def perm_body(jj, _, ios=ios, sh=sh):
            for u in range(_UNROLL):
                sl = pl.ds((jj * _UNROLL + u) * _NL, _NL)
                for (kin, pin, kout, pout, hist) in ios:
                    k = kin[sl]
                    d = lax.shift_right_logical(k, sh) & _DMASK
                    cnt, last = plsc.scan_count(d)
                    pos = plsc.load_gather(hist, [d]) + cnt - 1
                    plsc.store_scatter(kout, [pos], k)
                    plsc.store_scatter(pout, [pos], pin[sl])
                    plsc.store_scatter(hist, [d], pos + 1, mask=last)
            return 0

        lax.fori_loop(0, NV // _UNROLL, perm_body, 0)

    fin_a = len(_SHIFTS) % 2 == 0

    big = jnp.int32(S + 10)

    def lab_body(jj, carry):
        carry = list(carry)
        for u in range(_UNROLL):
            j = jj * _UNROLL + u
            sl = pl.ds(j * _NL, _NL)
            p_vec = lax.iota(jnp.int32, _NL) + j * _NL
            for i, r in enumerate(rows):
                kfin = r[1] if fin_a else r[3]
                pfin = r[2] if fin_a else r[4]
                obuf = r[6]
                uu = kfin[sl]
                flip = jnp.where(uu < 0, jnp.int32(-(2 ** 31)), jnp.int32(-1))
                v = plsc.bitcast(uu ^ flip, jnp.float32)
                x = jnp.clip(v + 0.5, 2.0, 4096.0)
                a = x.astype(jnp.int32) - p_vec
                a = jnp.where(p_vec >= 2, a, big)
                g = jnp.minimum(-plsc.cummax(-a), carry[i])
                lab = (p_vec + jnp.minimum(g, 1)).astype(jnp.float32)
                plsc.store_scatter(obuf, [pfin[sl]], lab)
                carry[i] = g[15]
        return tuple(carry)

    lax.fori_loop(0, NV // _UNROLL, lab_body, (big,) * _NR)


def kernel(norms):
    B, S = norms.shape
    mesh = plsc.VectorSubcoreMesh(core_axis_name="c", subcore_axis_name="s")
    n_workers = mesh.num_cores * mesh.num_subcores
    rows_per_w = B // n_workers

    scratch = []
    for _ in range(_NR):
        scratch += [
            pltpu.VMEM((S,), jnp.float32),
            pltpu.VMEM((S,), jnp.int32),
            pltpu.VMEM((S,), jnp.int32),
            pltpu.VMEM((S,), jnp.int32),
            pltpu.VMEM((S,), jnp.int32),
            pltpu.VMEM((_RADIX,), jnp.int32),
            pltpu.VMEM((S,), jnp.float32),
        ]

    @functools.partial(
        pl.kernel,
        out_type=jax.ShapeDtypeStruct((B, S), jnp.float32),
        mesh=mesh,
        scratch_types=scratch,
        compiler_params=pltpu.CompilerParams(needs_layout_passes=False),
    )
    def sc_kernel(norms_hbm, out_hbm, *bufs):
        rows = [tuple(bufs[7 * i:7 * i + 7]) for i in range(_NR)]
        wid = lax.axis_index("s") * mesh.num_cores + lax.axis_index("c")
        row0 = wid * rows_per_w

        def row_body(r, _):
            row = row0 + r * _NR
            for i in range(_NR):
                pltpu.sync_copy(norms_hbm.at[row + i], rows[i][0])
            _rows_sort_and_label(rows, S)
            for i in range(_NR):
                pltpu.sync_copy(rows[i][6], out_hbm.at[row + i])
            return 0

        lax.fori_loop(0, rows_per_w // _NR, row_body, 0)

    return sc_kernel(norms)


# speedup vs baseline: 2.9303x; 1.0183x over previous
import functools

import jax
import jax.numpy as jnp
from jax import lax
from jax.experimental import pallas as pl
from jax.experimental.pallas import tpu as pltpu
from jax.experimental.pallas import tpu_sc as plsc

_NL = 16
_RADIX = 2048
_DMASK = _RADIX - 1
_SHIFTS = (0, 11, 22)
_NR = 4
_UNROLL = 4


def _rows_sort_and_label(rows, S):
    NV = S // _NL
    NH = _RADIX // _NL

    def key_init(jj, _):
        for u in range(_UNROLL):
            j = jj * _UNROLL + u
            sl = pl.ds(j * _NL, _NL)
            for (vbuf, key_a, pay_a, _kb, _pb, _h, _o) in rows:
                b = plsc.bitcast(vbuf[sl], jnp.int32)
                flip = jnp.where(b < 0, jnp.int32(-1), jnp.int32(-(2 ** 31)))
                key_a[sl] = b ^ flip
                pay_a[sl] = lax.iota(jnp.int32, _NL) + j * _NL
        return 0

    lax.fori_loop(0, NV // _UNROLL, key_init, 0)

    for pidx, sh in enumerate(_SHIFTS):
        if pidx % 2 == 0:
            ios = [(r[1], r[2], r[3], r[4], r[5]) for r in rows]
        else:
            ios = [(r[3], r[4], r[1], r[2], r[5]) for r in rows]

        def zero_body(cc, _, ios=ios):
            for u in range(_UNROLL):
                sl = pl.ds((cc * _UNROLL + u) * _NL, _NL)
                for (_ki, _pi, _ko, _po, hist) in ios:
                    hist[sl] = jnp.zeros((_NL,), jnp.int32)
            return 0

        lax.fori_loop(0, NH // _UNROLL, zero_body, 0)

        def hist_body(jj, _, ios=ios, sh=sh):
            for u in range(_UNROLL):
                sl = pl.ds((jj * _UNROLL + u) * _NL, _NL)
                for (kin, _pi, _ko, _po, hist) in ios:
                    d = lax.shift_right_logical(kin[sl], sh) & _DMASK
                    cnt, last = plsc.scan_count(d)
                    plsc.addupdate_scatter(hist, [d], cnt, mask=last)
            return 0

        lax.fori_loop(0, NV // _UNROLL, hist_body, 0)

        def scan_body(cc, carry, ios=ios):
            carry = list(carry)
            for u in range(_UNROLL):
                sl = pl.ds((cc * _UNROLL + u) * _NL, _NL)
                for i, (_ki, _pi, _ko, _po, hist) in enumerate(ios):
                    h = hist[sl]
                    cs = plsc.cumsum(h)
                    hist[sl] = cs - h + carry[i]
                    carry[i] = carry[i] + cs[15]
            return tuple(carry)

        lax.fori_loop(0, NH // _UNROLL, scan_body, (jnp.int32(0),) * _NR)

        def perm_body(jj, _, ios=ios, sh=sh):
            for u in range(_UNROLL):
                sl = pl.ds((jj * _UNROLL + u) * _NL, _NL)
                for (kin, pin, kout, pout, hist) in ios:
                    k = kin[sl]
                    d = lax.shift_right_logical(k, sh) & _DMASK
                    cnt, last = plsc.scan_count(d)
                    pos = plsc.load_gather(hist, [d]) + cnt - 1
                    plsc.store_scatter(kout, [pos], k)
                    plsc.store_scatter(pout, [pos], pin[sl])
                    plsc.store_scatter(hist, [d], pos + 1, mask=last)
            return 0

        lax.fori_loop(0, NV // _UNROLL, perm_body, 0)

    fin_a = len(_SHIFTS) % 2 == 0

    big = jnp.int32(S + 10)

    def lab_body(jj, carry):
        carry = list(carry)
        for u in range(_UNROLL):
            j = jj * _UNROLL + u
            sl = pl.ds(j * _NL, _NL)
            p_vec = lax.iota(jnp.int32, _NL) + j * _NL
            for i, r in enumerate(rows):
                kfin = r[1] if fin_a else r[3]
                pfin = r[2] if fin_a else r[4]
                obuf = r[6]
                uu = kfin[sl]
                flip = jnp.where(uu < 0, jnp.int32(-(2 ** 31)), jnp.int32(-1))
                v = plsc.bitcast(uu ^ flip, jnp.float32)
                x = jnp.clip(v + 0.5, 2.0, 4096.0)
                a = x.astype(jnp.int32) - p_vec
                a = jnp.where(p_vec >= 2, a, big)
                g = jnp.minimum(-plsc.cummax(-a), carry[i])
                lab = (p_vec + jnp.minimum(g, 1)).astype(jnp.float32)
                plsc.store_scatter(obuf, [pfin[sl]], lab)
                carry[i] = g[15]
        return tuple(carry)

    lax.fori_loop(0, NV // _UNROLL, lab_body, (big,) * _NR)


def kernel(norms):
    B, S = norms.shape
    mesh = plsc.VectorSubcoreMesh(core_axis_name="c", subcore_axis_name="s")
    n_workers = mesh.num_cores * mesh.num_subcores
    rows_per_w = B // n_workers

    scratch = []
    for _ in range(_NR):
        scratch += [
            pltpu.VMEM((S,), jnp.float32),
            pltpu.VMEM((S,), jnp.int32),
            pltpu.VMEM((S,), jnp.int32),
            pltpu.VMEM((S,), jnp.int32),
            pltpu.VMEM((S,), jnp.int32),
            pltpu.VMEM((_RADIX,), jnp.int32),
            pltpu.VMEM((S,), jnp.float32),
        ]

    @functools.partial(
        pl.kernel,
        out_type=jax.ShapeDtypeStruct((B, S), jnp.float32),
        mesh=mesh,
        scratch_types=scratch,
        compiler_params=pltpu.CompilerParams(needs_layout_passes=False),
    )
    def sc_kernel(norms_hbm, out_hbm, *bufs):
        rows = [tuple(bufs[7 * i:7 * i + 7]) for i in range(_NR)]
        wid = lax.axis_index("s") * mesh.num_cores + lax.axis_index("c")
        row0 = wid * rows_per_w

        def row_body(r, _):
            row = row0 + r * _NR
            for i in range(_NR):
                pltpu.sync_copy(norms_hbm.at[row + i], rows[i][0])
            _rows_sort_and_label(rows, S)
            for i in range(_NR):
                pltpu.sync_copy(rows[i][6], out_hbm.at[row + i])
            return 0

        lax.fori_loop(0, rows_per_w // _NR, row_body, 0)

    return sc_kernel(norms)


# speedup vs baseline: 4.9063x; 1.6743x over previous
import functools

import jax
import jax.numpy as jnp
from jax import lax
from jax.experimental import pallas as pl
from jax.experimental.pallas import tpu as pltpu
from jax.experimental.pallas import tpu_sc as plsc

_NL = 16
_RADIX = 2048
_DMASK = _RADIX - 1
_SHIFTS = (0, 11, 22)
_NR = 4
_UNROLL = 4


def _rows_sort_and_label(rows, S):
    NV = S // _NL
    NH = _RADIX // _NL

    @plsc.parallel_loop(0, NV, unroll=_UNROLL)
    def key_init(j):
        sl = pl.ds(j * _NL, _NL)
        for (vbuf, key_a, pay_a, _kb, _pb, _h, _o) in rows:
            b = plsc.bitcast(vbuf[sl], jnp.int32)
            flip = jnp.where(b < 0, jnp.int32(-1), jnp.int32(-(2 ** 31)))
            key_a[sl] = b ^ flip
            pay_a[sl] = lax.iota(jnp.int32, _NL) + j * _NL

    for pidx, sh in enumerate(_SHIFTS):
        if pidx % 2 == 0:
            ios = [(r[1], r[2], r[3], r[4], r[5]) for r in rows]
        else:
            ios = [(r[3], r[4], r[1], r[2], r[5]) for r in rows]

        @plsc.parallel_loop(0, NH, unroll=_UNROLL)
        def zero_body(c, ios=ios):
            sl = pl.ds(c * _NL, _NL)
            for (_ki, _pi, _ko, _po, hist) in ios:
                hist[sl] = jnp.zeros((_NL,), jnp.int32)

        @plsc.parallel_loop(0, NV, unroll=_UNROLL)
        def hist_body(j, ios=ios, sh=sh):
            sl = pl.ds(j * _NL, _NL)
            for (kin, _pi, _ko, _po, hist) in ios:
                d = lax.shift_right_logical(kin[sl], sh) & _DMASK
                cnt, last = plsc.scan_count(d)
                plsc.addupdate_scatter(hist, [d], cnt, mask=last)

        def scan_body(cc, carry, ios=ios):
            carry = list(carry)
            for u in range(_UNROLL):
                sl = pl.ds((cc * _UNROLL + u) * _NL, _NL)
                for i, (_ki, _pi, _ko, _po, hist) in enumerate(ios):
                    h = hist[sl]
                    cs = plsc.cumsum(h)
                    hist[sl] = cs - h + carry[i]
                    carry[i] = carry[i] + cs[15]
            return tuple(carry)

        lax.fori_loop(0, NH // _UNROLL, scan_body, (jnp.int32(0),) * _NR)

        def perm_body(jj, _, ios=ios, sh=sh):
            for u in range(_UNROLL):
                sl = pl.ds((jj * _UNROLL + u) * _NL, _NL)
                for (kin, pin, kout, pout, hist) in ios:
                    k = kin[sl]
                    d = lax.shift_right_logical(k, sh) & _DMASK
                    cnt, last = plsc.scan_count(d)
                    pos = plsc.load_gather(hist, [d]) + cnt - 1
                    plsc.store_scatter(kout, [pos], k)
                    plsc.store_scatter(pout, [pos], pin[sl])
                    plsc.store_scatter(hist, [d], pos + 1, mask=last)
            return 0

        lax.fori_loop(0, NV // _UNROLL, perm_body, 0)

    fin_a = len(_SHIFTS) % 2 == 0
    big = jnp.int32(S + 10)

    @plsc.parallel_loop(0, NV, unroll=_UNROLL)
    def lab_local(j):
        sl = pl.ds(j * _NL, _NL)
        p_vec = lax.iota(jnp.int32, _NL) + j * _NL
        for r in rows:
            kfin = r[1] if fin_a else r[3]
            abuf = r[3] if fin_a else r[1]
            uu = kfin[sl]
            flip = jnp.where(uu < 0, jnp.int32(-(2 ** 31)), jnp.int32(-1))
            v = plsc.bitcast(uu ^ flip, jnp.float32)
            x = jnp.clip(v + 0.5, 2.0, 4096.0)
            a = x.astype(jnp.int32) - p_vec
            a = jnp.where(p_vec >= 2, a, big)
            abuf[sl] = -plsc.cummax(-a)

    def lab_prefix(c, carry):
        carry = list(carry)
        tail_idx = lax.iota(jnp.int32, _NL) * _NL + (c * _NL * _NL + _NL - 1)
        dst_idx = lax.iota(jnp.int32, _NL) + (c * _NL + 1)
        for i, r in enumerate(rows):
            abuf = r[3] if fin_a else r[1]
            pbuf = r[4] if fin_a else r[2]
            mins = plsc.load_gather(abuf, [tail_idx])
            incl = jnp.minimum(-plsc.cummax(-mins), carry[i])
            plsc.store_scatter(pbuf, [dst_idx], incl)
            carry[i] = incl[15]
        return tuple(carry)

    lax.fori_loop(0, NV // _NL, lab_prefix, (big,) * _NR)

    @plsc.parallel_loop(0, NV, unroll=_UNROLL)
    def lab_emit(j):
        sl = pl.ds(j * _NL, _NL)
        p_vec = lax.iota(jnp.int32, _NL) + j * _NL
        for r in rows:
            abuf = r[3] if fin_a else r[1]
            pbuf = r[4] if fin_a else r[2]
            pfin = r[2] if fin_a else r[4]
            obuf = r[6]
            pre = plsc.load_gather(pbuf, [jnp.where(j >= 1, j, 0)
                                          + jnp.zeros((_NL,), jnp.int32)])
            pre = jnp.where(j >= 1, pre, big)
            g = jnp.minimum(abuf[sl], pre)
            lab = (p_vec + jnp.minimum(g, 1)).astype(jnp.float32)
            plsc.store_scatter(obuf, [pfin[sl]], lab)


def kernel(norms):
    B, S = norms.shape
    mesh = plsc.VectorSubcoreMesh(core_axis_name="c", subcore_axis_name="s")
    n_workers = mesh.num_cores * mesh.num_subcores
    rows_per_w = B // n_workers

    scratch = []
    for _ in range(_NR):
        scratch += [
            pltpu.VMEM((S,), jnp.float32),
            pltpu.VMEM((S,), jnp.int32),
            pltpu.VMEM((S,), jnp.int32),
            pltpu.VMEM((S,), jnp.int32),
            pltpu.VMEM((S,), jnp.int32),
            pltpu.VMEM((_RADIX,), jnp.int32),
            pltpu.VMEM((S,), jnp.float32),
        ]

    @functools.partial(
        pl.kernel,
        out_type=jax.ShapeDtypeStruct((B, S), jnp.float32),
        mesh=mesh,
        scratch_types=scratch,
        compiler_params=pltpu.CompilerParams(needs_layout_passes=False),
    )
    def sc_kernel(norms_hbm, out_hbm, *bufs):
        rows = [tuple(bufs[7 * i:7 * i + 7]) for i in range(_NR)]
        wid = lax.axis_index("s") * mesh.num_cores + lax.axis_index("c")
        row0 = wid * rows_per_w

        def row_body(r, _):
            row = row0 + r * _NR
            for i in range(_NR):
                pltpu.sync_copy(norms_hbm.at[row + i], rows[i][0])
            _rows_sort_and_label(rows, S)
            for i in range(_NR):
                pltpu.sync_copy(rows[i][6], out_hbm.at[row + i])
            return 0

        lax.fori_loop(0, rows_per_w // _NR, row_body, 0)

    return sc_kernel(norms)
